# baseline (speedup 1.0000x reference)
import functools

import numpy as np
import jax
import jax.numpy as jnp
from jax import lax
from jax.experimental import pallas as pl
from jax.experimental.pallas import tpu as pltpu

_B, _C, _H, _W = 16, 3, 512, 512
_HS, _WS = 128, 128
_CIN = 2 * _C
_E, _K = 8, 2
_COUT = 16
_HWS = _HS * _WS
_INPUT_SIZE = _CIN * _HWS
_EPS = float(np.finfo(np.float64).eps)
_PIX_T = 2048


def _resize_matrix(in_size: int, out_size: int) -> np.ndarray:
    scale = out_size / in_size
    inv_scale = 1.0 / scale
    kernel_scale = max(inv_scale, 1.0)
    sample_f = (np.arange(out_size, dtype=np.float64) + 0.5) * inv_scale - 0.5
    x = np.abs(sample_f[np.newaxis, :]
               - np.arange(in_size, dtype=np.float64)[:, np.newaxis]) / kernel_scale
    w = np.maximum(0.0, 1.0 - x)
    total = np.sum(w, axis=0, keepdims=True)
    safe_total = np.where(total != 0, total, 1.0)
    w = np.where(np.abs(total) > 1000.0 * np.finfo(np.float32).eps, w / safe_total, 0.0)
    keep = (sample_f >= -0.5) & (sample_f <= in_size - 0.5)
    w = np.where(keep[np.newaxis, :], w, 0.0)
    return np.ascontiguousarray(w.T.astype(np.float32))


_RH = _resize_matrix(_H, _HS)
_RWT = np.ascontiguousarray(_resize_matrix(_W, _WS).T)


def _resize_body(src_ref, bgr_ref, rh_ref, rwt_ref, x_ref):
    rh = rh_ref[...]
    rwt = rwt_ref[...]
    s = src_ref[0]
    b = bgr_ref[0]
    for c in range(_C):
        t = jnp.dot(rh, s[c], preferred_element_type=jnp.float32, precision=lax.Precision.HIGHEST)
        x_ref[0, c] = jnp.dot(t, rwt, preferred_element_type=jnp.float32, precision=lax.Precision.HIGHEST)
        t = jnp.dot(rh, b[c], preferred_element_type=jnp.float32, precision=lax.Precision.HIGHEST)
        x_ref[0, _C + c] = jnp.dot(t, rwt, preferred_element_type=jnp.float32, precision=lax.Precision.HIGHEST)


def _logits_body(gx_ref, wg_ref, out_ref):
    out_ref[...] = jnp.dot(gx_ref[...], wg_ref[...],
                           preferred_element_type=jnp.float32)


def _gating_body(lg_ref, gates_ref, loss_ref):
    l = lg_ref[...]
    col = lax.broadcasted_iota(jnp.int32, (_B, _E), 1)
    m1 = jnp.max(l, axis=1, keepdims=True)
    i1 = jnp.min(jnp.where(l == m1, col, _E), axis=1, keepdims=True)
    mask1 = col == i1
    l2 = jnp.where(mask1, -jnp.inf, l)
    m2 = jnp.max(l2, axis=1, keepdims=True)
    i2 = jnp.min(jnp.where(l2 == m2, col, _E), axis=1, keepdims=True)
    mask2 = col == i2
    e2 = jnp.exp(m2 - m1)
    denom = 1.0 + e2
    g1 = 1.0 / denom
    g2 = e2 / denom
    gates = jnp.where(mask1, g1, 0.0) + jnp.where(mask2, g2, 0.0)
    gates_ref[...] = gates

    imp = jnp.sum(gates, axis=0)
    load = jnp.sum((gates > 0.0).astype(jnp.float32), axis=0)

    def cv2(v):
        mean = jnp.mean(v)
        var = jnp.sum((v - mean) ** 2) / (_E - 1)
        return var / (mean * mean + 1e-10)

    loss_ref[...] = ((cv2(imp) + cv2(load)) * 0.01).reshape(1, 1)


def _combine_body(x_ref, w_ref, g_ref, o_ref):
    xb = x_ref[0]
    w = w_ref[...]
    val = jnp.exp(jnp.dot(w, xb, preferred_element_type=jnp.float32))
    val = val.astype(jnp.bfloat16).astype(jnp.float32)
    g = g_ref[0].astype(jnp.bfloat16).astype(jnp.float32)
    acc = None
    for e in range(_E):
        part = val[e * _COUT:(e + 1) * _COUT, :] * g[0:1, e:e + 1]
        acc = part if acc is None else acc + part
    acc = jnp.where(acc == 0.0, _EPS, acc)
    o_ref[0] = jnp.log(acc)


def kernel(src, bgr, w_gate, expert_w):
    rh = jnp.asarray(_RH)
    rwt = jnp.asarray(_RWT)

    x = pl.pallas_call(
        _resize_body,
        grid=(_B,),
        in_specs=[
            pl.BlockSpec((1, _C, _H, _W), lambda i: (i, 0, 0, 0)),
            pl.BlockSpec((1, _C, _H, _W), lambda i: (i, 0, 0, 0)),
            pl.BlockSpec((_HS, _H), lambda i: (0, 0)),
            pl.BlockSpec((_W, _WS), lambda i: (0, 0)),
        ],
        out_specs=pl.BlockSpec((1, _CIN, _HS, _WS), lambda i: (i, 0, 0, 0)),
        out_shape=jax.ShapeDtypeStruct((_B, _CIN, _HS, _WS), jnp.float32),
    )(src, bgr, rh, rwt)

    gx = x.reshape(_B, _INPUT_SIZE)
    logits = pl.pallas_call(
        _logits_body,
        out_shape=jax.ShapeDtypeStruct((_B, _E), jnp.float32),
    )(gx, w_gate)

    gates, loss = pl.pallas_call(
        _gating_body,
        out_shape=(
            jax.ShapeDtypeStruct((_B, _E), jnp.float32),
            jax.ShapeDtypeStruct((1, 1), jnp.float32),
        ),
    )(logits)

    xf = x.reshape(_B, _CIN, _HWS)
    w_all = expert_w.reshape(_E * _COUT, _CIN)
    n_t = _HWS // _PIX_T
    out = pl.pallas_call(
        _combine_body,
        grid=(_B, n_t),
        in_specs=[
            pl.BlockSpec((1, _CIN, _PIX_T), lambda b, t: (b, 0, t)),
            pl.BlockSpec((_E * _COUT, _CIN), lambda b, t: (0, 0)),
            pl.BlockSpec((1, 1, _E), lambda b, t: (b, 0, 0)),
        ],
        out_specs=pl.BlockSpec((1, _COUT, _PIX_T), lambda b, t: (b, 0, t)),
        out_shape=jax.ShapeDtypeStruct((_B, _COUT, _HWS), jnp.float32),
    )(xf, w_all, gates.reshape(_B, 1, _E))

    return out.reshape(_B, _COUT, _HS, _WS), loss.reshape(())


# speedup vs baseline: 2.1669x; 2.1669x over previous
import functools

import numpy as np
import jax
import jax.numpy as jnp
from jax import lax
from jax.experimental import pallas as pl
from jax.experimental.pallas import tpu as pltpu
from jax.experimental.pallas import tpu_sc as plsc

_B, _C, _H, _W = 16, 3, 512, 512
_HS, _WS = 128, 128
_CIN = 2 * _C
_E, _K = 8, 2
_COUT = 16
_HWS = _HS * _WS
_INPUT_SIZE = _CIN * _HWS
_EPS = float(np.finfo(np.float64).eps)


def _resize_matrix(in_size: int, out_size: int) -> np.ndarray:
    scale = out_size / in_size
    inv_scale = 1.0 / scale
    kernel_scale = max(inv_scale, 1.0)
    sample_f = (np.arange(out_size, dtype=np.float64) + 0.5) * inv_scale - 0.5
    x = np.abs(sample_f[np.newaxis, :]
               - np.arange(in_size, dtype=np.float64)[:, np.newaxis]) / kernel_scale
    w = np.maximum(0.0, 1.0 - x)
    total = np.sum(w, axis=0, keepdims=True)
    safe_total = np.where(total != 0, total, 1.0)
    w = np.where(np.abs(total) > 1000.0 * np.finfo(np.float32).eps, w / safe_total, 0.0)
    keep = (sample_f >= -0.5) & (sample_f <= in_size - 0.5)
    w = np.where(keep[np.newaxis, :], w, 0.0)
    return np.ascontiguousarray(w.T.astype(np.float32))


_RH = _resize_matrix(_H, _HS)
_RWT = np.ascontiguousarray(_resize_matrix(_W, _WS).T)
_TAPS_LO = [(int(c), float(_RH[0, c])) for c in np.nonzero(_RH[0])[0]]
_TAPS_HI = [(int(c), float(_RH[_HS - 1, c])) for c in np.nonzero(_RH[_HS - 1])[0]]


def _split3(p):
    hi = p.astype(jnp.bfloat16)
    r1 = p - hi.astype(jnp.float32)
    mid = r1.astype(jnp.bfloat16)
    lo = (r1 - mid.astype(jnp.float32)).astype(jnp.bfloat16)
    return hi, mid, lo


def _resize_plane(p, rh, rwt):
    hi, mid, lo = _split3(p)
    t = (jnp.dot(rh, hi, preferred_element_type=jnp.float32)
         + jnp.dot(rh, mid, preferred_element_type=jnp.float32)
         + jnp.dot(rh, lo, preferred_element_type=jnp.float32))
    row0 = sum(w * p[i:i + 1, :] for i, w in _TAPS_LO)
    rowN = sum(w * p[i:i + 1, :] for i, w in _TAPS_HI)
    t = jnp.concatenate([row0, t[1:_HS - 1], rowN], axis=0)
    thi, tmid, tlo = _split3(t)
    y = (jnp.dot(thi, rwt, preferred_element_type=jnp.float32)
         + jnp.dot(tmid, rwt, preferred_element_type=jnp.float32)
         + jnp.dot(tlo, rwt, preferred_element_type=jnp.float32))
    col0 = sum(w * t[:, i:i + 1] for i, w in _TAPS_LO)
    colN = sum(w * t[:, i:i + 1] for i, w in _TAPS_HI)
    return jnp.concatenate([col0, y[:, 1:_WS - 1], colN], axis=1)


def _resize_body(src_ref, bgr_ref, rh_ref, rwt_ref, wgp_ref, x_ref, lg_ref):
    rh = rh_ref[...]
    rwt = rwt_ref[...]
    planes = []
    for half, ref in ((0, src_ref), (1, bgr_ref)):
        for c in range(_C):
            y = _resize_plane(ref[0, c], rh, rwt)
            x_ref[0, half * _C + c] = y
            planes.append(y.astype(jnp.bfloat16).astype(jnp.float32))
    row = []
    for e in range(_E):
        acc = None
        for ci in range(_CIN):
            part = planes[ci] * wgp_ref[e, ci].astype(jnp.float32)
            acc = part if acc is None else acc + part
        row.append(jnp.sum(acc).reshape(1, 1, 1))
    lg_ref[...] = jnp.concatenate(row, axis=2)


def _gating_sc_body(lg_hbm, idx_hbm, g_hbm, lg_v, idx_v, g_v):
    wid = lax.axis_index("s") * 2 + lax.axis_index("c")

    @pl.when(wid == 0)
    def _():
        pltpu.sync_copy(lg_hbm, lg_v)
        rows = [lg_v[e, :] for e in range(_E)]
        m1 = rows[0]
        i1 = jnp.zeros((16,), jnp.int32)
        for e in range(1, _E):
            better = rows[e] > m1
            m1 = jnp.where(better, rows[e], m1)
            i1 = jnp.where(better, jnp.full((16,), e, jnp.int32), i1)
        neg_inf = jnp.full((16,), -jnp.inf, jnp.float32)
        m2 = neg_inf
        i2 = jnp.zeros((16,), jnp.int32)
        for e in range(_E):
            cand = jnp.where(i1 == jnp.full((16,), e, jnp.int32), neg_inf, rows[e])
            better = cand > m2
            m2 = jnp.where(better, cand, m2)
            i2 = jnp.where(better, jnp.full((16,), e, jnp.int32), i2)
        e2 = jnp.exp(m2 - m1)
        denom = 1.0 + e2
        g1 = 1.0 / denom
        g2 = e2 / denom
        idx_v[0, :] = i1
        idx_v[1, :] = i2
        g_v[0, :] = g1
        g_v[1, :] = g2
        pltpu.sync_copy(idx_v, idx_hbm)
        pltpu.sync_copy(g_v, g_hbm)


def _combine_body(idx_ref, x_ref, w1_ref, w2_ref, g_ref, idxa_ref, ga_ref, o_ref, loss_ref):
    @pl.when(pl.program_id(0) == 0)
    def _():
        col = lax.broadcasted_iota(jnp.int32, (_B, _E), 1)
        ia = idxa_ref[...]
        ga = ga_ref[0]
        gates = (jnp.where(col == ia[:, 0:1], ga[:, 0:1], 0.0)
                 + jnp.where(col == ia[:, 1:2], ga[:, 1:2], 0.0))
        imp = jnp.sum(gates, axis=0)
        load = jnp.sum((gates > 0.0).astype(jnp.float32), axis=0)

        def cv2(v):
            mean = jnp.mean(v)
            var = jnp.sum((v - mean) ** 2) / (_E - 1)
            return var / (mean * mean + 1e-10)

        loss_ref[...] = ((cv2(imp) + cv2(load)) * 0.01).reshape(1, 1)

    xb = x_ref[0].reshape(_CIN, _HWS)
    eo1 = jnp.dot(w1_ref[0], xb, preferred_element_type=jnp.float32)
    eo2 = jnp.dot(w2_ref[0], xb, preferred_element_type=jnp.float32)
    v1 = jnp.exp(eo1).astype(jnp.bfloat16).astype(jnp.float32)
    v2 = jnp.exp(eo2).astype(jnp.bfloat16).astype(jnp.float32)
    g = g_ref[0, 0].astype(jnp.bfloat16).astype(jnp.float32)
    acc = v1 * g[0:1].reshape(1, 1) + v2 * g[1:2].reshape(1, 1)
    acc = jnp.where(acc == 0.0, _EPS, acc)
    o_ref[0] = jnp.log(acc).reshape(_COUT, _HS, _WS)


def _resize(src, bgr, w_gate):
    rh = jnp.asarray(_RH, dtype=jnp.bfloat16)
    rwt = jnp.asarray(_RWT, dtype=jnp.bfloat16)
    wgp = w_gate.T.reshape(_E, _CIN, _HS, _WS).astype(jnp.bfloat16)
    return pl.pallas_call(
        _resize_body,
        grid=(_B,),
        in_specs=[
            pl.BlockSpec((1, _C, _H, _W), lambda i: (i, 0, 0, 0)),
            pl.BlockSpec((1, _C, _H, _W), lambda i: (i, 0, 0, 0)),
            pl.BlockSpec((_HS, _H), lambda i: (0, 0)),
            pl.BlockSpec((_W, _WS), lambda i: (0, 0)),
            pl.BlockSpec((_E, _CIN, _HS, _WS), lambda i: (0, 0, 0, 0)),
        ],
        out_specs=(
            pl.BlockSpec((1, _CIN, _HS, _WS), lambda i: (i, 0, 0, 0)),
            pl.BlockSpec((1, 1, _E), lambda i: (i, 0, 0)),
        ),
        out_shape=(
            jax.ShapeDtypeStruct((_B, _CIN, _HS, _WS), jnp.float32),
            jax.ShapeDtypeStruct((_B, 1, _E), jnp.float32),
        ),
    )(src, bgr, rh, rwt, wgp)


def _gating(lgT):
    mesh = plsc.VectorSubcoreMesh(core_axis_name="c", subcore_axis_name="s")
    run = functools.partial(
        pl.kernel,
        mesh=mesh,
        out_type=(
            jax.ShapeDtypeStruct((_K, _B), jnp.int32),
            jax.ShapeDtypeStruct((_K, _B), jnp.float32),
        ),
        scratch_types=[
            pltpu.VMEM((_E, _B), jnp.float32),
            pltpu.VMEM((_K, _B), jnp.int32),
            pltpu.VMEM((_K, _B), jnp.float32),
        ],
    )(_gating_sc_body)
    return run(lgT)


def _combine(idx, x, expert_w, g12):
    return pl.pallas_call(
        _combine_body,
        grid_spec=pltpu.PrefetchScalarGridSpec(
            num_scalar_prefetch=1,
            grid=(_B,),
            in_specs=[
                pl.BlockSpec((1, _CIN, _HS, _WS), lambda b, idx_ref: (b, 0, 0, 0)),
                pl.BlockSpec((1, _COUT, _CIN), lambda b, idx_ref: (idx_ref[b, 0], 0, 0)),
                pl.BlockSpec((1, _COUT, _CIN), lambda b, idx_ref: (idx_ref[b, 1], 0, 0)),
                pl.BlockSpec((1, 1, _K), lambda b, idx_ref: (b, 0, 0)),
                pl.BlockSpec((_B, _K), lambda b, idx_ref: (0, 0)),
                pl.BlockSpec((1, _B, _K), lambda b, idx_ref: (0, 0, 0)),
            ],
            out_specs=(
                pl.BlockSpec((1, _COUT, _HS, _WS), lambda b, idx_ref: (b, 0, 0, 0)),
                pl.BlockSpec((1, 1), lambda b, idx_ref: (0, 0)),
            ),
        ),
        out_shape=(
            jax.ShapeDtypeStruct((_B, _COUT, _HS, _WS), jnp.float32),
            jax.ShapeDtypeStruct((1, 1), jnp.float32),
        ),
    )(idx, x, expert_w, expert_w, g12.reshape(_B, 1, _K), idx,
      g12.reshape(1, _B, _K))


def kernel(src, bgr, w_gate, expert_w):
    x, lg = _resize(src, bgr, w_gate)
    idxT, gT = _gating(lg.reshape(_B, _E).T)
    out, loss = _combine(idxT.T, x, expert_w, gT.T)
    return out, loss.reshape(())


# speedup vs baseline: 2.2806x; 1.0525x over previous
import functools

import numpy as np
import jax
import jax.numpy as jnp
from jax import lax
from jax.experimental import pallas as pl
from jax.experimental.pallas import tpu as pltpu
from jax.experimental.pallas import tpu_sc as plsc

_B, _C, _H, _W = 16, 3, 512, 512
_HS, _WS = 128, 128
_CIN = 2 * _C
_E, _K = 8, 2
_COUT = 16
_HWS = _HS * _WS
_INPUT_SIZE = _CIN * _HWS
_EPS = float(np.finfo(np.float64).eps)


def _resize_matrix(in_size: int, out_size: int) -> np.ndarray:
    scale = out_size / in_size
    inv_scale = 1.0 / scale
    kernel_scale = max(inv_scale, 1.0)
    sample_f = (np.arange(out_size, dtype=np.float64) + 0.5) * inv_scale - 0.5
    x = np.abs(sample_f[np.newaxis, :]
               - np.arange(in_size, dtype=np.float64)[:, np.newaxis]) / kernel_scale
    w = np.maximum(0.0, 1.0 - x)
    total = np.sum(w, axis=0, keepdims=True)
    safe_total = np.where(total != 0, total, 1.0)
    w = np.where(np.abs(total) > 1000.0 * np.finfo(np.float32).eps, w / safe_total, 0.0)
    keep = (sample_f >= -0.5) & (sample_f <= in_size - 0.5)
    w = np.where(keep[np.newaxis, :], w, 0.0)
    return np.ascontiguousarray(w.T.astype(np.float32))


_RH = _resize_matrix(_H, _HS)
_RWT = np.ascontiguousarray(_resize_matrix(_W, _WS).T)
_TAPS_LO = [(int(c), float(_RH[0, c])) for c in np.nonzero(_RH[0])[0]]
_TAPS_HI = [(int(c), float(_RH[_HS - 1, c])) for c in np.nonzero(_RH[_HS - 1])[0]]


def _split3(p):
    hi = p.astype(jnp.bfloat16)
    r1 = p - hi.astype(jnp.float32)
    mid = r1.astype(jnp.bfloat16)
    lo = (r1 - mid.astype(jnp.float32)).astype(jnp.bfloat16)
    return hi, mid, lo


def _resize_plane(p, rh, rwt):
    hi, mid, lo = _split3(p)
    t = (jnp.dot(rh, hi, preferred_element_type=jnp.float32)
         + jnp.dot(rh, mid, preferred_element_type=jnp.float32)
         + jnp.dot(rh, lo, preferred_element_type=jnp.float32))
    row0 = sum(w * p[i:i + 1, :] for i, w in _TAPS_LO)
    rowN = sum(w * p[i:i + 1, :] for i, w in _TAPS_HI)
    t = jnp.concatenate([row0, t[1:_HS - 1], rowN], axis=0)
    thi, tmid, tlo = _split3(t)
    y = (jnp.dot(thi, rwt, preferred_element_type=jnp.float32)
         + jnp.dot(tmid, rwt, preferred_element_type=jnp.float32)
         + jnp.dot(tlo, rwt, preferred_element_type=jnp.float32))
    col0 = sum(w * t[:, i:i + 1] for i, w in _TAPS_LO)
    colN = sum(w * t[:, i:i + 1] for i, w in _TAPS_HI)
    return jnp.concatenate([col0, y[:, 1:_WS - 1], colN], axis=1)


def _resize_body(src_ref, bgr_ref, rh_ref, rwt_ref, wgp_ref, x_ref, lg_ref):
    rh = rh_ref[...]
    rwt = rwt_ref[...]
    planes = []
    for half, ref in ((0, src_ref), (1, bgr_ref)):
        for c in range(_C):
            y = _resize_plane(ref[0, c], rh, rwt)
            x_ref[0, half * _C + c] = y
            planes.append(y.astype(jnp.bfloat16).astype(jnp.float32))
    col = []
    for e in range(_E):
        acc = None
        for ci in range(_CIN):
            part = planes[ci] * wgp_ref[e, ci].astype(jnp.float32)
            acc = part if acc is None else acc + part
        col.append(jnp.sum(acc).reshape(1, 1))
    b = pl.program_id(0)
    col8 = jnp.concatenate(col, axis=0)
    lane = lax.broadcasted_iota(jnp.int32, (_E, _B), 1)
    lg_ref[...] = jnp.where(lane == b, col8, lg_ref[...])


def _gating_sc_body(lg_hbm, idx_hbm, g_hbm, lg_v, idx_v, g_v):
    wid = lax.axis_index("s") * 2 + lax.axis_index("c")

    @pl.when(wid == 0)
    def _():
        pltpu.sync_copy(lg_hbm, lg_v)
        rows = [lg_v[e, :] for e in range(_E)]
        m1 = rows[0]
        i1 = jnp.zeros((16,), jnp.int32)
        for e in range(1, _E):
            better = rows[e] > m1
            m1 = jnp.where(better, rows[e], m1)
            i1 = jnp.where(better, jnp.full((16,), e, jnp.int32), i1)
        neg_inf = jnp.full((16,), -jnp.inf, jnp.float32)
        m2 = neg_inf
        i2 = jnp.zeros((16,), jnp.int32)
        for e in range(_E):
            cand = jnp.where(i1 == jnp.full((16,), e, jnp.int32), neg_inf, rows[e])
            better = cand > m2
            m2 = jnp.where(better, cand, m2)
            i2 = jnp.where(better, jnp.full((16,), e, jnp.int32), i2)
        e2 = jnp.exp(m2 - m1)
        denom = 1.0 + e2
        g1 = 1.0 / denom
        g2 = e2 / denom
        idx_v[0, :] = i1
        idx_v[1, :] = i2
        g_v[0, :] = g1
        g_v[1, :] = g2
        pltpu.sync_copy(idx_v, idx_hbm)
        pltpu.sync_copy(g_v, g_hbm)


def _combine_body(idx_ref, idxv_ref, x_ref, w1_ref, w2_ref, gt_ref, o_ref, loss_ref):
    b = pl.program_id(0)

    @pl.when(b == 0)
    def _():
        ioe = lax.broadcasted_iota(jnp.int32, (_E, _B), 0)
        ia = idxv_ref[...]
        ga = gt_ref[...]
        gates = (jnp.where(ioe == ia[0:1, :], ga[0:1, :], 0.0)
                 + jnp.where(ioe == ia[1:2, :], ga[1:2, :], 0.0))
        imp = jnp.sum(gates, axis=1)
        load = jnp.sum((gates > 0.0).astype(jnp.float32), axis=1)

        def cv2(v):
            mean = jnp.mean(v)
            var = jnp.sum((v - mean) ** 2) / (_E - 1)
            return var / (mean * mean + 1e-10)

        loss_ref[...] = ((cv2(imp) + cv2(load)) * 0.01).reshape(1, 1)

    xb = x_ref[0].reshape(_CIN, _HWS)
    eo1 = jnp.dot(w1_ref[0], xb, preferred_element_type=jnp.float32)
    eo2 = jnp.dot(w2_ref[0], xb, preferred_element_type=jnp.float32)
    v1 = jnp.exp(eo1).astype(jnp.bfloat16).astype(jnp.float32)
    v2 = jnp.exp(eo2).astype(jnp.bfloat16).astype(jnp.float32)
    onehot = (lax.broadcasted_iota(jnp.int32, (_K, _B), 1) == b).astype(jnp.float32)
    gb = gt_ref[...].astype(jnp.bfloat16).astype(jnp.float32)
    gsel = jnp.sum(gb * onehot, axis=1, keepdims=True)
    acc = v1 * gsel[0:1, :] + v2 * gsel[1:2, :]
    acc = jnp.where(acc == 0.0, _EPS, acc)
    o_ref[0] = jnp.log(acc).reshape(_COUT, _HS, _WS)


def _resize(src, bgr, w_gate):
    rh = jnp.asarray(_RH, dtype=jnp.bfloat16)
    rwt = jnp.asarray(_RWT, dtype=jnp.bfloat16)
    wgp = w_gate.T.reshape(_E, _CIN, _HS, _WS).astype(jnp.bfloat16)
    return pl.pallas_call(
        _resize_body,
        grid=(_B,),
        in_specs=[
            pl.BlockSpec((1, _C, _H, _W), lambda i: (i, 0, 0, 0)),
            pl.BlockSpec((1, _C, _H, _W), lambda i: (i, 0, 0, 0)),
            pl.BlockSpec((_HS, _H), lambda i: (0, 0)),
            pl.BlockSpec((_W, _WS), lambda i: (0, 0)),
            pl.BlockSpec((_E, _CIN, _HS, _WS), lambda i: (0, 0, 0, 0)),
        ],
        out_specs=(
            pl.BlockSpec((1, _CIN, _HS, _WS), lambda i: (i, 0, 0, 0)),
            pl.BlockSpec((_E, _B), lambda i: (0, 0)),
        ),
        out_shape=(
            jax.ShapeDtypeStruct((_B, _CIN, _HS, _WS), jnp.float32),
            jax.ShapeDtypeStruct((_E, _B), jnp.float32),
        ),
    )(src, bgr, rh, rwt, wgp)


def _gating(lgT):
    mesh = plsc.VectorSubcoreMesh(core_axis_name="c", subcore_axis_name="s")
    run = functools.partial(
        pl.kernel,
        mesh=mesh,
        out_type=(
            jax.ShapeDtypeStruct((_K, _B), jnp.int32),
            jax.ShapeDtypeStruct((_K, _B), jnp.float32),
        ),
        scratch_types=[
            pltpu.VMEM((_E, _B), jnp.float32),
            pltpu.VMEM((_K, _B), jnp.int32),
            pltpu.VMEM((_K, _B), jnp.float32),
        ],
    )(_gating_sc_body)
    return run(lgT)


def _combine(idxT, x, expert_w, gT):
    return pl.pallas_call(
        _combine_body,
        grid_spec=pltpu.PrefetchScalarGridSpec(
            num_scalar_prefetch=1,
            grid=(_B,),
            in_specs=[
                pl.BlockSpec((_K, _B), lambda b, idx_ref: (0, 0)),
                pl.BlockSpec((1, _CIN, _HS, _WS), lambda b, idx_ref: (b, 0, 0, 0)),
                pl.BlockSpec((1, _COUT, _CIN), lambda b, idx_ref: (idx_ref[0, b], 0, 0)),
                pl.BlockSpec((1, _COUT, _CIN), lambda b, idx_ref: (idx_ref[1, b], 0, 0)),
                pl.BlockSpec((_K, _B), lambda b, idx_ref: (0, 0)),
            ],
            out_specs=(
                pl.BlockSpec((1, _COUT, _HS, _WS), lambda b, idx_ref: (b, 0, 0, 0)),
                pl.BlockSpec((1, 1), lambda b, idx_ref: (0, 0)),
            ),
        ),
        out_shape=(
            jax.ShapeDtypeStruct((_B, _COUT, _HS, _WS), jnp.float32),
            jax.ShapeDtypeStruct((1, 1), jnp.float32),
        ),
    )(idxT, idxT, x, expert_w, expert_w, gT)


def kernel(src, bgr, w_gate, expert_w):
    x, lgT = _resize(src, bgr, w_gate)
    idxT, gT = _gating(lgT)
    out, loss = _combine(idxT, x, expert_w, gT)
    return out, loss.reshape(())


# speedup vs baseline: 2.5774x; 1.1301x over previous
import functools

import numpy as np
import jax
import jax.numpy as jnp
from jax import lax
from jax.experimental import pallas as pl
from jax.experimental.pallas import tpu as pltpu

_B, _C, _H, _W = 16, 3, 512, 512
_HS, _WS = 128, 128
_CIN = 2 * _C
_E, _K = 8, 2
_COUT = 16
_HWS = _HS * _WS
_INPUT_SIZE = _CIN * _HWS
_EPS = float(np.finfo(np.float64).eps)


def _resize_matrix(in_size: int, out_size: int) -> np.ndarray:
    scale = out_size / in_size
    inv_scale = 1.0 / scale
    kernel_scale = max(inv_scale, 1.0)
    sample_f = (np.arange(out_size, dtype=np.float64) + 0.5) * inv_scale - 0.5
    x = np.abs(sample_f[np.newaxis, :]
               - np.arange(in_size, dtype=np.float64)[:, np.newaxis]) / kernel_scale
    w = np.maximum(0.0, 1.0 - x)
    total = np.sum(w, axis=0, keepdims=True)
    safe_total = np.where(total != 0, total, 1.0)
    w = np.where(np.abs(total) > 1000.0 * np.finfo(np.float32).eps, w / safe_total, 0.0)
    keep = (sample_f >= -0.5) & (sample_f <= in_size - 0.5)
    w = np.where(keep[np.newaxis, :], w, 0.0)
    return np.ascontiguousarray(w.T.astype(np.float32))


_RH = _resize_matrix(_H, _HS)
_RWT = np.ascontiguousarray(_resize_matrix(_W, _WS).T)
_TAPS_LO = [(int(c), float(_RH[0, c])) for c in np.nonzero(_RH[0])[0]]
_TAPS_HI = [(int(c), float(_RH[_HS - 1, c])) for c in np.nonzero(_RH[_HS - 1])[0]]


def _split3(p):
    hi = p.astype(jnp.bfloat16)
    r1 = p - hi.astype(jnp.float32)
    mid = r1.astype(jnp.bfloat16)
    lo = (r1 - mid.astype(jnp.float32)).astype(jnp.bfloat16)
    return hi, mid, lo


def _resize_plane(p, rh, rwt):
    hi, mid, lo = _split3(p)
    t = (jnp.dot(rh, hi, preferred_element_type=jnp.float32)
         + jnp.dot(rh, mid, preferred_element_type=jnp.float32)
         + jnp.dot(rh, lo, preferred_element_type=jnp.float32))
    row0 = sum(w * p[i:i + 1, :] for i, w in _TAPS_LO)
    rowN = sum(w * p[i:i + 1, :] for i, w in _TAPS_HI)
    t = jnp.concatenate([row0, t[1:_HS - 1], rowN], axis=0)
    thi, tmid, tlo = _split3(t)
    y = (jnp.dot(thi, rwt, preferred_element_type=jnp.float32)
         + jnp.dot(tmid, rwt, preferred_element_type=jnp.float32)
         + jnp.dot(tlo, rwt, preferred_element_type=jnp.float32))
    col0 = sum(w * t[:, i:i + 1] for i, w in _TAPS_LO)
    colN = sum(w * t[:, i:i + 1] for i, w in _TAPS_HI)
    return jnp.concatenate([col0, y[:, 1:_WS - 1], colN], axis=1)


def _moe_body(src_ref, bgr_ref, rh_ref, rwt_ref, wgp_ref, wf_ref,
              o_ref, loss_ref, gsc_ref):
    b = pl.program_id(0)
    rh = rh_ref[...]
    rwt = rwt_ref[...]

    planes = []
    planes_b16 = []
    for half, ref in ((0, src_ref), (1, bgr_ref)):
        for c in range(_C):
            y = _resize_plane(ref[0, c], rh, rwt)
            planes.append(y)
            planes_b16.append(y.astype(jnp.bfloat16).astype(jnp.float32))

    row = []
    for e in range(_E):
        acc = None
        for ci in range(_CIN):
            part = planes_b16[ci] * wgp_ref[e, ci].astype(jnp.float32)
            acc = part if acc is None else acc + part
        row.append(jnp.sum(acc).reshape(1, 1))
    l = jnp.concatenate(row, axis=1)

    col = lax.broadcasted_iota(jnp.int32, (1, _E), 1)
    m1 = jnp.max(l, axis=1, keepdims=True)
    i1 = jnp.min(jnp.where(l == m1, col, _E), axis=1, keepdims=True)
    mask1 = col == i1
    l2 = jnp.where(mask1, -jnp.inf, l)
    m2 = jnp.max(l2, axis=1, keepdims=True)
    i2 = jnp.min(jnp.where(l2 == m2, col, _E), axis=1, keepdims=True)
    mask2 = col == i2
    e2 = jnp.exp(m2 - m1)
    denom = 1.0 + e2
    g1 = 1.0 / denom
    g2 = e2 / denom
    gates_row = jnp.where(mask1, g1, 0.0) + jnp.where(mask2, g2, 0.0)

    rowio = lax.broadcasted_iota(jnp.int32, (_B, _E), 0)
    gsc_ref[...] = jnp.where(rowio == b, gates_row, gsc_ref[...])

    m1f = mask1.astype(jnp.float32)
    m2f = mask2.astype(jnp.float32)
    w1 = None
    w2 = None
    for e in range(_E):
        we = wf_ref[e].reshape(_COUT, _CIN)
        p1 = we * m1f[0:1, e:e + 1]
        p2 = we * m2f[0:1, e:e + 1]
        w1 = p1 if w1 is None else w1 + p1
        w2 = p2 if w2 is None else w2 + p2

    xb = jnp.concatenate([p.reshape(1, _HS, _WS) for p in planes],
                         axis=0).reshape(_CIN, _HWS)
    eo1 = jnp.dot(w1, xb, preferred_element_type=jnp.float32)
    eo2 = jnp.dot(w2, xb, preferred_element_type=jnp.float32)
    v1 = jnp.exp(eo1).astype(jnp.bfloat16).astype(jnp.float32)
    v2 = jnp.exp(eo2).astype(jnp.bfloat16).astype(jnp.float32)
    gb1 = g1.astype(jnp.bfloat16).astype(jnp.float32)
    gb2 = g2.astype(jnp.bfloat16).astype(jnp.float32)
    acc = v1 * gb1 + v2 * gb2
    acc = jnp.where(acc == 0.0, _EPS, acc)
    o_ref[0] = jnp.log(acc).reshape(_COUT, _HS, _WS)

    @pl.when(b == _B - 1)
    def _():
        gates = gsc_ref[...]
        imp = jnp.sum(gates, axis=0)
        load = jnp.sum((gates > 0.0).astype(jnp.float32), axis=0)

        def cv2(v):
            mean = jnp.mean(v)
            var = jnp.sum((v - mean) ** 2) / (_E - 1)
            return var / (mean * mean + 1e-10)

        loss_ref[...] = ((cv2(imp) + cv2(load)) * 0.01).reshape(1, 1)


def kernel(src, bgr, w_gate, expert_w):
    rh = jnp.asarray(_RH, dtype=jnp.bfloat16)
    rwt = jnp.asarray(_RWT, dtype=jnp.bfloat16)
    wgp = w_gate.T.reshape(_E, _CIN, _HS, _WS).astype(jnp.bfloat16)
    wf = expert_w
    out, loss = pl.pallas_call(
        _moe_body,
        grid=(_B,),
        in_specs=[
            pl.BlockSpec((1, _C, _H, _W), lambda i: (i, 0, 0, 0)),
            pl.BlockSpec((1, _C, _H, _W), lambda i: (i, 0, 0, 0)),
            pl.BlockSpec((_HS, _H), lambda i: (0, 0)),
            pl.BlockSpec((_W, _WS), lambda i: (0, 0)),
            pl.BlockSpec((_E, _CIN, _HS, _WS), lambda i: (0, 0, 0, 0)),
            pl.BlockSpec((_E, _COUT, _CIN), lambda i: (0, 0, 0)),
        ],
        out_specs=(
            pl.BlockSpec((1, _COUT, _HS, _WS), lambda i: (i, 0, 0, 0)),
            pl.BlockSpec((1, 1), lambda i: (0, 0)),
        ),
        out_shape=(
            jax.ShapeDtypeStruct((_B, _COUT, _HS, _WS), jnp.float32),
            jax.ShapeDtypeStruct((1, 1), jnp.float32),
        ),
        scratch_shapes=[pltpu.VMEM((_B, _E), jnp.float32)],
    )(src, bgr, rh, rwt, wgp, wf)
    return out, loss.reshape(())
